# baseline (speedup 1.0000x reference)
import functools

import numpy as np

import jax
import jax.numpy as jnp
from jax import lax
from jax.experimental import pallas as pl
from jax.experimental.pallas import tpu as pltpu
from jax.experimental.pallas import tpu_sc as plsc

_NC = 2
_NS = 16
_NW = _NC * _NS
_L = 16
_CH = 80
_CPB = 5
_NBUF = 3


def _scores_body(rt_ref, pw_ref, out_ref):
    s = lax.dot_general(rt_ref[...].astype(jnp.float32), pw_ref[...],
                        (((1,), (1,)), ((), ())),
                        preferred_element_type=jnp.float32)
    out_ref[...] = jax.nn.sigmoid(s)


def _compute_scores(review_table, prob_score_w):
    R, D = review_table.shape
    RB = 10000
    return pl.pallas_call(
        _scores_body,
        grid=(R // RB,),
        in_specs=[pl.BlockSpec((RB, D), lambda i: (i, 0)),
                  pl.BlockSpec((1, D), lambda i: (0, 0))],
        out_specs=pl.BlockSpec((RB, 1), lambda i: (i, 0)),
        out_shape=jax.ShapeDtypeStruct((R, 1), jnp.float32),
    )(review_table, prob_score_w)


def _make_agg_kernel(N, D, E):
    per_tile = E // _NW
    nchunk = per_tile // _CH
    nblk = nchunk // _CPB
    rows_per_tile = (N // _NS) // 8 * 8
    tail_rows = N - rows_per_tile * _NS
    mesh = plsc.VectorSubcoreMesh(core_axis_name="c", subcore_axis_name="s",
                                  num_cores=_NC, num_subcores=_NS)

    @functools.partial(
        pl.kernel,
        out_type=jax.ShapeDtypeStruct((_NC, N, D), jnp.float32),
        mesh=mesh,
        scratch_types=[
            pltpu.VMEM((_CPB, _CH), jnp.int32),
            pltpu.VMEM((_CPB, _CH), jnp.int32),
            pltpu.VMEM((_CPB, _CH), jnp.int32),
            pltpu.VMEM((_NBUF, _CH), jnp.float32),
            pltpu.VMEM((_NBUF, _CH), jnp.float32),
            pltpu.VMEM((_CH + _L,), jnp.float32),
            pltpu.VMEM((_NBUF, _CH, D), jnp.float32),
            pltpu.VMEM_SHARED((N, D), jnp.float32),
            pltpu.SemaphoreType.DMA,
        ] + [pltpu.SemaphoreType.DMA] * (2 * _NBUF),
    )
    def agg_kernel(feat_hbm, ei_hbm, rid_hbm, sig_hbm, cj_hbm,
                   zero_hbm, out_hbm,
                   srcb, dstb, ridb, sv, cv, wv, rows, agg,
                   isem, *sems):
        c = lax.axis_index("c")
        s = lax.axis_index("s")
        wid = c * _NS + s
        row0 = s * rows_per_tile
        pltpu.sync_copy(zero_hbm.at[pl.ds(row0, rows_per_tile)],
                        agg.at[pl.ds(row0, rows_per_tile)])
        if tail_rows:
            @pl.when(s == 0)
            def _zero_tail():
                t0 = rows_per_tile * _NS
                pltpu.sync_copy(zero_hbm.at[pl.ds(t0, tail_rows)],
                                agg.at[pl.ds(t0, tail_rows)])
        plsc.subcore_barrier()

        gsems = sems[:_NBUF]
        ssems = sems[_NBUF:]
        half = _CH // 2

        def fire_gathers(k, p):
            gs = gsems[p]
            return (
                pltpu.async_copy(feat_hbm.at[srcb.at[k, pl.ds(0, half)]],
                                 rows.at[p, pl.ds(0, half)], gs),
                pltpu.async_copy(feat_hbm.at[srcb.at[k, pl.ds(half, half)]],
                                 rows.at[p, pl.ds(half, half)], gs),
                pltpu.async_copy(sig_hbm.at[ridb.at[k]], sv.at[p], gs),
                pltpu.async_copy(cj_hbm.at[srcb.at[k]], cv.at[p], gs),
            )

        def scale_and_fire_scatter(k, p):
            for q in range(_CH // _L):
                sl = pl.ds(q * _L, _L)
                wv[sl] = sv[p, sl] * cv[p, sl]

            def scale_edge(e, carry2):
                wvec = wv[pl.ds(e, _L)]
                ws = lax.gather(
                    wvec, jnp.zeros((_L, 1), jnp.int32),
                    lax.GatherDimensionNumbers(offset_dims=(),
                                               collapsed_slice_dims=(0,),
                                               start_index_map=(0,)),
                    slice_sizes=(1,),
                    mode=lax.GatherScatterMode.PROMISE_IN_BOUNDS)
                for j in range(D // _L):
                    slj = pl.ds(j * _L, _L)
                    rows[p, e, slj] = rows[p, e, slj] * ws
                return carry2

            lax.fori_loop(0, _CH, scale_edge, 0, unroll=2)
            return pltpu.async_copy(rows.at[p], agg.at[dstb.at[k]], ssems[p],
                                    add=True)

        def blk_body(b, carry):
            b3 = wid * nblk + b
            ci1 = pltpu.async_copy(ei_hbm.at[0, b3], srcb, isem)
            ci2 = pltpu.async_copy(ei_hbm.at[1, b3], dstb, isem)
            ci3 = pltpu.async_copy(rid_hbm.at[b3], ridb, isem)
            ci1.wait(); ci2.wait(); ci3.wait()

            gd = [None] * _NBUF
            sd = [None] * _NBUF
            for k0 in range(min(_NBUF, _CPB)):
                gd[k0] = fire_gathers(k0, k0)
            for k in range(_CPB):
                p = k % _NBUF
                for d in gd[p]:
                    d.wait()
                sd[p] = scale_and_fire_scatter(k, p)
                nxt = k + _NBUF
                if nxt < _CPB:
                    sd[p].wait()
                    gd[p] = fire_gathers(nxt, p)
            for k in range(max(0, _CPB - _NBUF), _CPB):
                sd[k % _NBUF].wait()
            return carry

        lax.fori_loop(0, nblk, blk_body, 0)

        plsc.subcore_barrier()
        pltpu.sync_copy(agg.at[pl.ds(row0, rows_per_tile)],
                        out_hbm.at[c, pl.ds(row0, rows_per_tile)])
        if tail_rows:
            @pl.when(s == 0)
            def _copy_tail():
                t0 = rows_per_tile * _NS
                pltpu.sync_copy(agg.at[pl.ds(t0, tail_rows)],
                                out_hbm.at[c, pl.ds(t0, tail_rows)])

    return agg_kernel


def _final_body(p0_ref, p1_ref, ci_ref, lw_ref, lb_ref, out_ref):
    a = (p0_ref[...] + p1_ref[...]) * ci_ref[...]
    out_ref[...] = lax.dot_general(a, lw_ref[...], (((1,), (1,)), ((), ())),
                                   preferred_element_type=jnp.float32) + lb_ref[...]


def _final(part0, part1, ci, linear_w, linear_b):
    N, D = part0.shape
    NB = 2000
    return pl.pallas_call(
        _final_body,
        grid=(N // NB,),
        in_specs=[pl.BlockSpec((NB, D), lambda i: (i, 0)),
                  pl.BlockSpec((NB, D), lambda i: (i, 0)),
                  pl.BlockSpec((NB, 1), lambda i: (i, 0)),
                  pl.BlockSpec((D, D), lambda i: (0, 0)),
                  pl.BlockSpec((1, D), lambda i: (0, 0))],
        out_specs=pl.BlockSpec((NB, D), lambda i: (i, 0)),
        out_shape=jax.ShapeDtypeStruct((N, D), jnp.float32),
    )(part0, part1, ci, linear_w, linear_b.reshape(1, D))


def kernel(feat, edge_index, review_id, cj, ci, review_table, prob_score_w,
           linear_w, linear_b):
    N, D = feat.shape
    E = edge_index.shape[1]
    R = review_table.shape[0]

    sig = _compute_scores(review_table.astype(jnp.bfloat16),
                          prob_score_w).reshape(R)


    ei = edge_index.reshape(2, -1, _CPB, _CH)
    rid = review_id.reshape(-1, _CPB, _CH)
    cjf = cj.reshape(N)
    zeros = jnp.zeros((N, D), jnp.float32)

    part = _make_agg_kernel(N, D, E)(feat, ei, rid, sig, cjf, zeros)

    return _final(part[0], part[1], ci, linear_w, linear_b)


# speedup vs baseline: 1.1401x; 1.1401x over previous
import functools

import numpy as np

import jax
import jax.numpy as jnp
from jax import lax
from jax.experimental import pallas as pl
from jax.experimental.pallas import tpu as pltpu
from jax.experimental.pallas import tpu_sc as plsc

_NC = 2
_NS = 16
_NW = _NC * _NS
_L = 16
_CH = 80
_CPB = 10
_NBUF = 2


def _scores_body(rt_ref, pw_ref, out_ref):
    s = lax.dot_general(rt_ref[...], pw_ref[...], (((1,), (1,)), ((), ())),
                        preferred_element_type=jnp.float32)
    out_ref[...] = jax.nn.sigmoid(s)


def _compute_scores(review_table, prob_score_w):
    R, D = review_table.shape
    RB = 10000
    return pl.pallas_call(
        _scores_body,
        grid=(R // RB,),
        in_specs=[pl.BlockSpec((RB, D), lambda i: (i, 0)),
                  pl.BlockSpec((1, D), lambda i: (0, 0))],
        out_specs=pl.BlockSpec((RB, 1), lambda i: (i, 0)),
        out_shape=jax.ShapeDtypeStruct((R, 1), jnp.float32),
    )(review_table, prob_score_w)


def _make_agg_kernel(N, D, E):
    per_tile = E // _NW
    nchunk = per_tile // _CH
    nblk = nchunk // _CPB
    rows_per_tile = (N // _NS) // 8 * 8
    tail_rows = N - rows_per_tile * _NS
    mesh = plsc.VectorSubcoreMesh(core_axis_name="c", subcore_axis_name="s",
                                  num_cores=_NC, num_subcores=_NS)

    @functools.partial(
        pl.kernel,
        out_type=jax.ShapeDtypeStruct((_NC, N, D), jnp.float32),
        mesh=mesh,
        scratch_types=[
            pltpu.VMEM((_CPB, _CH), jnp.int32),
            pltpu.VMEM((_CPB, _CH), jnp.int32),
            pltpu.VMEM((_CPB, _CH), jnp.int32),
            pltpu.VMEM((_NBUF, _CH), jnp.float32),
            pltpu.VMEM((_NBUF, _CH), jnp.float32),
            pltpu.VMEM((_CH + _L,), jnp.float32),
            pltpu.VMEM((_NBUF, _CH, D), jnp.float32),
            pltpu.VMEM_SHARED((N, D), jnp.float32),
            pltpu.SemaphoreType.DMA,
        ] + [pltpu.SemaphoreType.DMA] * (2 * _NBUF),
    )
    def agg_kernel(feat_hbm, ei_hbm, rid_hbm, sig_hbm, cj_hbm,
                   zero_hbm, out_hbm,
                   srcb, dstb, ridb, sv, cv, wv, rows, agg,
                   isem, *sems):
        c = lax.axis_index("c")
        s = lax.axis_index("s")
        wid = c * _NS + s
        row0 = s * rows_per_tile
        pltpu.sync_copy(zero_hbm.at[pl.ds(row0, rows_per_tile)],
                        agg.at[pl.ds(row0, rows_per_tile)])
        if tail_rows:
            @pl.when(s == 0)
            def _zero_tail():
                t0 = rows_per_tile * _NS
                pltpu.sync_copy(zero_hbm.at[pl.ds(t0, tail_rows)],
                                agg.at[pl.ds(t0, tail_rows)])
        plsc.subcore_barrier()

        gsems = sems[:_NBUF]
        ssems = sems[_NBUF:]
        half = _CH // 2

        def fire_gathers(k, p):
            gs = gsems[p]
            return (
                pltpu.async_copy(feat_hbm.at[srcb.at[k, pl.ds(0, half)]],
                                 rows.at[p, pl.ds(0, half)], gs),
                pltpu.async_copy(feat_hbm.at[srcb.at[k, pl.ds(half, half)]],
                                 rows.at[p, pl.ds(half, half)], gs),
                pltpu.async_copy(sig_hbm.at[ridb.at[k]], sv.at[p], gs),
                pltpu.async_copy(cj_hbm.at[srcb.at[k]], cv.at[p], gs),
            )

        def scale_and_fire_scatter(k, p):
            for q in range(_CH // _L):
                sl = pl.ds(q * _L, _L)
                wv[sl] = sv[p, sl] * cv[p, sl]

            def scale_edge(e, carry2):
                wvec = wv[pl.ds(e, _L)]
                ws = lax.gather(
                    wvec, jnp.zeros((_L, 1), jnp.int32),
                    lax.GatherDimensionNumbers(offset_dims=(),
                                               collapsed_slice_dims=(0,),
                                               start_index_map=(0,)),
                    slice_sizes=(1,),
                    mode=lax.GatherScatterMode.PROMISE_IN_BOUNDS)
                for j in range(D // _L):
                    slj = pl.ds(j * _L, _L)
                    rows[p, e, slj] = rows[p, e, slj] * ws
                return carry2

            lax.fori_loop(0, _CH, scale_edge, 0, unroll=2)
            return pltpu.async_copy(rows.at[p], agg.at[dstb.at[k]], ssems[p],
                                    add=True)

        def blk_body(b, carry):
            b3 = wid * nblk + b
            ci1 = pltpu.async_copy(ei_hbm.at[0, b3], srcb, isem)
            ci2 = pltpu.async_copy(ei_hbm.at[1, b3], dstb, isem)
            ci3 = pltpu.async_copy(rid_hbm.at[b3], ridb, isem)
            ci1.wait(); ci2.wait(); ci3.wait()

            gd = [None] * _NBUF
            sd = [None] * _NBUF
            for k0 in range(min(_NBUF, _CPB)):
                gd[k0] = fire_gathers(k0, k0)
            for k in range(_CPB):
                p = k % _NBUF
                for d in gd[p]:
                    d.wait()
                sd[p] = scale_and_fire_scatter(k, p)
                nxt = k + _NBUF
                if nxt < _CPB:
                    sd[p].wait()
                    gd[p] = fire_gathers(nxt, p)
            for k in range(max(0, _CPB - _NBUF), _CPB):
                sd[k % _NBUF].wait()
            return carry

        lax.fori_loop(0, nblk, blk_body, 0)

        plsc.subcore_barrier()
        pltpu.sync_copy(agg.at[pl.ds(row0, rows_per_tile)],
                        out_hbm.at[c, pl.ds(row0, rows_per_tile)])
        if tail_rows:
            @pl.when(s == 0)
            def _copy_tail():
                t0 = rows_per_tile * _NS
                pltpu.sync_copy(agg.at[pl.ds(t0, tail_rows)],
                                out_hbm.at[c, pl.ds(t0, tail_rows)])

    return agg_kernel


def _final_body(p0_ref, p1_ref, ci_ref, lw_ref, lb_ref, out_ref):
    a = (p0_ref[...] + p1_ref[...]) * ci_ref[...]
    out_ref[...] = lax.dot_general(a, lw_ref[...], (((1,), (1,)), ((), ())),
                                   preferred_element_type=jnp.float32) + lb_ref[...]


def _final(part0, part1, ci, linear_w, linear_b):
    N, D = part0.shape
    NB = 2000
    return pl.pallas_call(
        _final_body,
        grid=(N // NB,),
        in_specs=[pl.BlockSpec((NB, D), lambda i: (i, 0)),
                  pl.BlockSpec((NB, D), lambda i: (i, 0)),
                  pl.BlockSpec((NB, 1), lambda i: (i, 0)),
                  pl.BlockSpec((D, D), lambda i: (0, 0)),
                  pl.BlockSpec((1, D), lambda i: (0, 0))],
        out_specs=pl.BlockSpec((NB, D), lambda i: (i, 0)),
        out_shape=jax.ShapeDtypeStruct((N, D), jnp.float32),
    )(part0, part1, ci, linear_w, linear_b.reshape(1, D))


def kernel(feat, edge_index, review_id, cj, ci, review_table, prob_score_w,
           linear_w, linear_b):
    N, D = feat.shape
    E = edge_index.shape[1]
    R = review_table.shape[0]

    sig = _compute_scores(review_table, prob_score_w).reshape(R)


    ei = edge_index.reshape(2, -1, _CPB, _CH)
    rid = review_id.reshape(-1, _CPB, _CH)
    cjf = cj.reshape(N)
    zeros = jnp.zeros((N, D), jnp.float32)

    part = _make_agg_kernel(N, D, E)(feat, ei, rid, sig, cjf, zeros)

    return _final(part[0], part[1], ci, linear_w, linear_b)


# speedup vs baseline: 1.1752x; 1.0308x over previous
import functools

import numpy as np

import jax
import jax.numpy as jnp
from jax import lax
from jax.experimental import pallas as pl
from jax.experimental.pallas import tpu as pltpu
from jax.experimental.pallas import tpu_sc as plsc

_NC = 2
_NS = 16
_NW = _NC * _NS
_L = 16
_CH = 80
_CPB = 25
_NBUF = 2


def _scores_body(rt_ref, pw_ref, out_ref):
    s = lax.dot_general(rt_ref[...], pw_ref[...], (((1,), (1,)), ((), ())),
                        preferred_element_type=jnp.float32)
    out_ref[...] = jax.nn.sigmoid(s)


def _compute_scores(review_table, prob_score_w):
    R, D = review_table.shape
    RB = 10000
    return pl.pallas_call(
        _scores_body,
        grid=(R // RB,),
        in_specs=[pl.BlockSpec((RB, D), lambda i: (i, 0)),
                  pl.BlockSpec((1, D), lambda i: (0, 0))],
        out_specs=pl.BlockSpec((RB, 1), lambda i: (i, 0)),
        out_shape=jax.ShapeDtypeStruct((R, 1), jnp.float32),
    )(review_table, prob_score_w)


def _make_agg_kernel(N, D, E):
    per_tile = E // _NW
    nchunk = per_tile // _CH
    nblk = nchunk // _CPB
    rows_per_tile = (N // _NS) // 8 * 8
    tail_rows = N - rows_per_tile * _NS
    mesh = plsc.VectorSubcoreMesh(core_axis_name="c", subcore_axis_name="s",
                                  num_cores=_NC, num_subcores=_NS)

    @functools.partial(
        pl.kernel,
        out_type=jax.ShapeDtypeStruct((_NC, N, D), jnp.float32),
        mesh=mesh,
        scratch_types=[
            pltpu.VMEM((_CPB, _CH), jnp.int32),
            pltpu.VMEM((_CPB, _CH), jnp.int32),
            pltpu.VMEM((_CPB, _CH), jnp.int32),
            pltpu.VMEM((_NBUF, _CH), jnp.float32),
            pltpu.VMEM((_NBUF, _CH), jnp.float32),
            pltpu.VMEM((_CH + _L,), jnp.float32),
            pltpu.VMEM((_NBUF, _CH, D), jnp.float32),
            pltpu.VMEM_SHARED((N, D), jnp.float32),
            pltpu.SemaphoreType.DMA,
        ] + [pltpu.SemaphoreType.DMA] * (2 * _NBUF),
    )
    def agg_kernel(feat_hbm, ei_hbm, rid_hbm, sig_hbm, cj_hbm,
                   zero_hbm, out_hbm,
                   srcb, dstb, ridb, sv, cv, wv, rows, agg,
                   isem, *sems):
        c = lax.axis_index("c")
        s = lax.axis_index("s")
        wid = c * _NS + s
        row0 = s * rows_per_tile
        pltpu.sync_copy(zero_hbm.at[pl.ds(row0, rows_per_tile)],
                        agg.at[pl.ds(row0, rows_per_tile)])
        if tail_rows:
            @pl.when(s == 0)
            def _zero_tail():
                t0 = rows_per_tile * _NS
                pltpu.sync_copy(zero_hbm.at[pl.ds(t0, tail_rows)],
                                agg.at[pl.ds(t0, tail_rows)])
        plsc.subcore_barrier()

        gsems = sems[:_NBUF]
        ssems = sems[_NBUF:]
        half = _CH // 2

        def fire_gathers(k, p):
            gs = gsems[p]
            return (
                pltpu.async_copy(feat_hbm.at[srcb.at[k, pl.ds(0, half)]],
                                 rows.at[p, pl.ds(0, half)], gs),
                pltpu.async_copy(feat_hbm.at[srcb.at[k, pl.ds(half, half)]],
                                 rows.at[p, pl.ds(half, half)], gs),
                pltpu.async_copy(sig_hbm.at[ridb.at[k]], sv.at[p], gs),
                pltpu.async_copy(cj_hbm.at[srcb.at[k]], cv.at[p], gs),
            )

        def scale_and_fire_scatter(k, p):
            for q in range(_CH // _L):
                sl = pl.ds(q * _L, _L)
                wv[sl] = sv[p, sl] * cv[p, sl]

            def scale_edge(e, carry2):
                wvec = wv[pl.ds(e, _L)]
                ws = lax.gather(
                    wvec, jnp.zeros((_L, 1), jnp.int32),
                    lax.GatherDimensionNumbers(offset_dims=(),
                                               collapsed_slice_dims=(0,),
                                               start_index_map=(0,)),
                    slice_sizes=(1,),
                    mode=lax.GatherScatterMode.PROMISE_IN_BOUNDS)
                for j in range(D // _L):
                    slj = pl.ds(j * _L, _L)
                    rows[p, e, slj] = rows[p, e, slj] * ws
                return carry2

            lax.fori_loop(0, _CH, scale_edge, 0, unroll=2)
            return pltpu.async_copy(rows.at[p], agg.at[dstb.at[k]], ssems[p],
                                    add=True)

        def blk_body(b, carry):
            b3 = wid * nblk + b
            ci1 = pltpu.async_copy(ei_hbm.at[0, b3], srcb, isem)
            ci2 = pltpu.async_copy(ei_hbm.at[1, b3], dstb, isem)
            ci3 = pltpu.async_copy(rid_hbm.at[b3], ridb, isem)
            ci1.wait(); ci2.wait(); ci3.wait()

            gd = [None] * _NBUF
            sd = [None] * _NBUF
            for k0 in range(min(_NBUF, _CPB)):
                gd[k0] = fire_gathers(k0, k0)
            for k in range(_CPB):
                p = k % _NBUF
                for d in gd[p]:
                    d.wait()
                sd[p] = scale_and_fire_scatter(k, p)
                nxt = k + _NBUF
                if nxt < _CPB:
                    sd[p].wait()
                    gd[p] = fire_gathers(nxt, p)
            for k in range(max(0, _CPB - _NBUF), _CPB):
                sd[k % _NBUF].wait()
            return carry

        lax.fori_loop(0, nblk, blk_body, 0)

        plsc.subcore_barrier()
        pltpu.sync_copy(agg.at[pl.ds(row0, rows_per_tile)],
                        out_hbm.at[c, pl.ds(row0, rows_per_tile)])
        if tail_rows:
            @pl.when(s == 0)
            def _copy_tail():
                t0 = rows_per_tile * _NS
                pltpu.sync_copy(agg.at[pl.ds(t0, tail_rows)],
                                out_hbm.at[c, pl.ds(t0, tail_rows)])

    return agg_kernel


def _final_body(p0_ref, p1_ref, ci_ref, lw_ref, lb_ref, out_ref):
    a = (p0_ref[...] + p1_ref[...]) * ci_ref[...]
    out_ref[...] = lax.dot_general(a, lw_ref[...], (((1,), (1,)), ((), ())),
                                   preferred_element_type=jnp.float32) + lb_ref[...]


def _final(part0, part1, ci, linear_w, linear_b):
    N, D = part0.shape
    NB = 2000
    return pl.pallas_call(
        _final_body,
        grid=(N // NB,),
        in_specs=[pl.BlockSpec((NB, D), lambda i: (i, 0)),
                  pl.BlockSpec((NB, D), lambda i: (i, 0)),
                  pl.BlockSpec((NB, 1), lambda i: (i, 0)),
                  pl.BlockSpec((D, D), lambda i: (0, 0)),
                  pl.BlockSpec((1, D), lambda i: (0, 0))],
        out_specs=pl.BlockSpec((NB, D), lambda i: (i, 0)),
        out_shape=jax.ShapeDtypeStruct((N, D), jnp.float32),
    )(part0, part1, ci, linear_w, linear_b.reshape(1, D))


def kernel(feat, edge_index, review_id, cj, ci, review_table, prob_score_w,
           linear_w, linear_b):
    N, D = feat.shape
    E = edge_index.shape[1]
    R = review_table.shape[0]

    sig = _compute_scores(review_table, prob_score_w).reshape(R)


    ei = edge_index.reshape(2, -1, _CPB, _CH)
    rid = review_id.reshape(-1, _CPB, _CH)
    cjf = cj.reshape(N)
    zeros = jnp.zeros((N, D), jnp.float32)

    part = _make_agg_kernel(N, D, E)(feat, ei, rid, sig, cjf, zeros)

    return _final(part[0], part[1], ci, linear_w, linear_b)
